# baseline (speedup 1.0000x reference)
import functools

import jax
import jax.numpy as jnp
from jax import lax
from jax.experimental import pallas as pl
from jax.experimental.pallas import tpu as pltpu
from jax.experimental.pallas import tpu_sc as plsc

_NUM_CLASSES = 20
_DSF = 8
_NC = 2
_NS = 16
_L = 16


def _sc_modal_downsample(labels, class_weights):
    bs, H, W = labels.shape
    Hs, Ws = H // _DSF, W // _DSF
    n_strips = bs * Hs
    strip_words = _DSF * W
    NW = _NC * _NS
    strips_per_w = n_strips // NW
    groups = Ws // _L
    labels_strips = labels.reshape(n_strips, strip_words)
    mesh = plsc.VectorSubcoreMesh(core_axis_name="c", subcore_axis_name="s")

    @functools.partial(
        pl.kernel,
        mesh=mesh,
        out_type=jax.ShapeDtypeStruct((n_strips, Ws), jnp.int32),
        compiler_params=pltpu.CompilerParams(needs_layout_passes=False),
        scratch_types=[
            pltpu.VMEM((strip_words,), jnp.int32),
            pltpu.VMEM((_L * _NUM_CLASSES,), jnp.int32),
            pltpu.VMEM((Ws,), jnp.int32),
            pltpu.VMEM((_NUM_CLASSES,), jnp.float32),
        ],
    )
    def body(labels_hbm, w_hbm, out_hbm, buf, hist, outrow, wbuf):
        wid = lax.axis_index("s") * _NC + lax.axis_index("c")
        iota = lax.iota(jnp.int32, _L)
        iota8 = iota * _DSF
        iota_h = iota * _NUM_CLASSES
        ones = jnp.ones((_L,), jnp.int32)
        zeros = jnp.zeros((_L,), jnp.int32)
        pltpu.sync_copy(w_hbm, wbuf)

        def strip_body(j, carry):
            s = wid * strips_per_w + j
            pltpu.sync_copy(labels_hbm.at[s], buf)
            for g in range(groups):
                for i in range(_L * _NUM_CLASSES // _L):
                    hist[pl.ds(i * _L, _L)] = zeros
                for re in range(_DSF):
                    for k in range(_DSF):
                        idx = iota8 + (re * W + g * _L * _DSF + k)
                        v = plsc.load_gather(buf, [idx])
                        plsc.addupdate_scatter(hist, [iota_h + v], ones)

                def cls_body(c, bc):
                    best_score, best_class = bc
                    cvec = jnp.full((_L,), c, jnp.int32)
                    cnt = plsc.load_gather(hist, [iota_h + cvec])
                    w = plsc.load_gather(wbuf, [cvec])
                    score = cnt.astype(jnp.float32) * w
                    upd = score > best_score
                    return (jnp.where(upd, score, best_score),
                            jnp.where(upd, cvec, best_class))

                init = (jnp.full((_L,), -jnp.inf, jnp.float32), zeros)
                _, best_class = lax.fori_loop(0, _NUM_CLASSES, cls_body, init)
                outrow[pl.ds(g * _L, _L)] = best_class
            pltpu.sync_copy(outrow, out_hbm.at[s])
            return carry

        lax.fori_loop(0, strips_per_w, strip_body, 0)

    out = body(labels_strips, class_weights)
    return out.reshape(bs, Hs, Ws)


def kernel(labels, class_weights, dsf):
    del dsf
    return _sc_modal_downsample(labels, class_weights)
